# Initial kernel scaffold; baseline (speedup 1.0000x reference)
import jax
import jax.numpy as jnp
from jax.experimental import pallas as pl


def kernel(feat, initial_feat, edge_index, W, l):
    raise NotImplementedError("write your pallas kernel here")


# speedup vs baseline: 3.8875x; 3.8875x over previous
import functools

import jax
import jax.numpy as jnp
from jax import lax
from jax.experimental import pallas as pl
from jax.experimental.pallas import tpu as pltpu
from jax.experimental.pallas import tpu_sc as plsc

N = 10000
D = 256
HALF = 128
NP = 10240
E = 160000
CHUNK = 128
NCH = 79
EPT = NCH * CHUNK
EPAD = 16 * EPT
AVPT = NCH * CHUNK
ALPHA = 0.1


def _deg_call(idx3, zeros_deg, ones_v_hbm):
    mesh = plsc.VectorSubcoreMesh(core_axis_name="c", subcore_axis_name="s")

    @functools.partial(
        pl.kernel,
        mesh=mesh,
        out_type=jax.ShapeDtypeStruct((2, 2 * NP), jnp.float32),
        scratch_types=[
            pltpu.VMEM((NCH, CHUNK), jnp.int32),
            pltpu.VMEM((CHUNK,), jnp.float32),
            pltpu.VMEM_SHARED((2 * NP,), jnp.float32),
        ],
    )
    def degk(idx_hbm, z_hbm, ones_hbm, out_hbm, idx_v, ones_v, acc):
        c = lax.axis_index("c")
        s = lax.axis_index("s")
        wid = s * 2 + c
        pltpu.sync_copy(idx_hbm.at[wid], idx_v)
        pltpu.sync_copy(ones_hbm, ones_v)
        sl = pl.ds(s * (2 * NP // 16), 2 * NP // 16)
        pltpu.sync_copy(z_hbm.at[sl], acc.at[sl])
        plsc.subcore_barrier()

        def body(j, _):
            pltpu.sync_copy(ones_v, acc.at[idx_v.at[j]], add=True)
            return 0

        lax.fori_loop(0, NCH, body, 0)
        plsc.subcore_barrier()
        pltpu.sync_copy(acc.at[sl], out_hbm.at[c, sl])

    return degk(idx3, zeros_deg, ones_v_hbm)


def _agg_call(h0, h1, src3, dst3, zeros_h):
    mesh = plsc.VectorSubcoreMesh(core_axis_name="c", subcore_axis_name="s")

    @functools.partial(
        pl.kernel,
        mesh=mesh,
        out_type=[
            jax.ShapeDtypeStruct((NP, HALF), jnp.float32),
            jax.ShapeDtypeStruct((NP, HALF), jnp.float32),
        ],
        scratch_types=[
            pltpu.VMEM((NCH, CHUNK), jnp.int32),
            pltpu.VMEM((NCH, CHUNK), jnp.int32),
            pltpu.VMEM((CHUNK, HALF), jnp.float32),
            pltpu.VMEM_SHARED((NP, HALF), jnp.float32),
            pltpu.SemaphoreType.DMA,
        ],
    )
    def aggk(h0_hbm, h1_hbm, src_hbm, dst_hbm, z_hbm, out0_hbm, out1_hbm,
             src_v, dst_v, buf, acc, sem):
        c = lax.axis_index("c")
        s = lax.axis_index("s")
        pltpu.sync_copy(src_hbm.at[s], src_v)
        pltpu.sync_copy(dst_hbm.at[s], dst_v)
        rows = pl.ds(s * (NP // 16), NP // 16)
        pltpu.sync_copy(z_hbm.at[rows], acc.at[rows])
        plsc.subcore_barrier()

        def run(h_hbm):
            def body(j, _):
                pltpu.async_copy(h_hbm.at[src_v.at[j]], buf, sem).wait()
                pltpu.sync_copy(buf, acc.at[dst_v.at[j]], add=True)
                return 0

            lax.fori_loop(0, NCH, body, 0)

        @pl.when(c == 0)
        def _():
            run(h0_hbm)

        @pl.when(c == 1)
        def _():
            run(h1_hbm)

        plsc.subcore_barrier()

        @pl.when(c == 0)
        def _():
            pltpu.sync_copy(acc.at[rows], out0_hbm.at[rows])

        @pl.when(c == 1)
        def _():
            pltpu.sync_copy(acc.at[rows], out1_hbm.at[rows])

    return aggk(h0, h1, src3, dst3, zeros_h)


def _h_body(feat_ref, deg_ref, h0_ref, h1_ref):
    dp = deg_ref[...]
    outd = jnp.maximum(dp[0, 0, :] + dp[1, 0, :], 1.0)
    ns = lax.rsqrt(outd)
    h = feat_ref[...] * ns[:, None]
    h0_ref[...] = h[:, :HALF]
    h1_ref[...] = h[:, HALF:]


def _h_call(feat, deg3):
    R = 512
    nb = NP // R
    return pl.pallas_call(
        _h_body,
        grid=(nb,),
        in_specs=[
            pl.BlockSpec((R, D), lambda i: (i, 0)),
            pl.BlockSpec((2, 2, R), lambda i: (0, 0, i)),
        ],
        out_specs=[
            pl.BlockSpec((R, HALF), lambda i: (i, 0)),
            pl.BlockSpec((R, HALF), lambda i: (i, 0)),
        ],
        out_shape=[
            jax.ShapeDtypeStruct((NP, HALF), jnp.float32),
            jax.ShapeDtypeStruct((NP, HALF), jnp.float32),
        ],
    )(feat, deg3)


def _out_body(beta_ref, r0_ref, r1_ref, deg_ref, init_ref, w_ref, out_ref):
    dp = deg_ref[...]
    ind = jnp.maximum(dp[0, 1, :] + dp[1, 1, :], 1.0)
    nd = lax.rsqrt(ind)[:, None]
    rst = jnp.concatenate([r0_ref[...], r1_ref[...]], axis=1) * nd
    a = (1.0 - ALPHA) * rst + ALPHA * init_ref[...]
    b = beta_ref[0, 0]
    out_ref[...] = b * jnp.dot(a, w_ref[...], preferred_element_type=jnp.float32) \
        + (1.0 - b) * a


def _out_call(beta, r0, r1, deg3, initial_feat, W):
    R = 512
    nb = NP // R
    return pl.pallas_call(
        _out_body,
        grid=(nb,),
        in_specs=[
            pl.BlockSpec(memory_space=pltpu.SMEM),
            pl.BlockSpec((R, HALF), lambda i: (i, 0)),
            pl.BlockSpec((R, HALF), lambda i: (i, 0)),
            pl.BlockSpec((2, 2, R), lambda i: (0, 0, i)),
            pl.BlockSpec((R, D), lambda i: (i, 0)),
            pl.BlockSpec((D, D), lambda i: (0, 0)),
        ],
        out_specs=pl.BlockSpec((R, D), lambda i: (i, 0)),
        out_shape=jax.ShapeDtypeStruct((NP, D), jnp.float32),
    )(beta, r0, r1, deg3, initial_feat, W)


def kernel(feat, initial_feat, edge_index, W, l):
    src = edge_index[0].astype(jnp.int32)
    dst = edge_index[1].astype(jnp.int32)

    idx_flat = jnp.concatenate([src, dst + NP])
    idx_flat = jnp.pad(idx_flat, (0, 32 * AVPT - 2 * E), constant_values=N)
    idx3 = idx_flat.reshape(32, NCH, CHUNK)

    src3 = jnp.pad(src, (0, EPAD - E), constant_values=N).reshape(16, NCH, CHUNK)
    dst3 = jnp.pad(dst, (0, EPAD - E), constant_values=N).reshape(16, NCH, CHUNK)

    zeros_deg = jnp.zeros((2 * NP,), jnp.float32)
    ones_v = jnp.ones((CHUNK,), jnp.float32)
    zeros_h = jnp.zeros((NP, HALF), jnp.float32)

    feat_pad = jnp.pad(feat, ((0, NP - N), (0, 0)))
    init_pad = jnp.pad(initial_feat, ((0, NP - N), (0, 0)))

    deg2 = _deg_call(idx3, zeros_deg, ones_v)
    deg3 = deg2.reshape(2, 2, NP)

    h0, h1 = _h_call(feat_pad, deg3)
    r0, r1 = _agg_call(h0, h1, src3, dst3, zeros_h)

    beta = jnp.log(0.5 / l + 1.0).astype(jnp.float32).reshape(1, 1)
    return _out_call(beta, r0, r1, deg3, init_pad, W)[:N]
